# Initial kernel scaffold; baseline (speedup 1.0000x reference)
import jax
import jax.numpy as jnp
from jax.experimental import pallas as pl


def kernel(table, rel_index):
    raise NotImplementedError("write your pallas kernel here")


# speedup vs baseline: 2.8281x; 2.8281x over previous
import functools

import jax
import jax.numpy as jnp
from jax import lax
from jax.experimental import pallas as pl
from jax.experimental.pallas import tpu as pltpu
from jax.experimental.pallas import tpu_sc as plsc

_N = 256
_NH = 32
_NN = _N * _N
_ROWS = 961
_NW = 32
_CHUNK = _NN // _NW
_L = 16


def _sc_gather(table, idx_flat):
    mesh = plsc.VectorSubcoreMesh(core_axis_name="c", subcore_axis_name="s")

    @functools.partial(
        pl.kernel,
        mesh=mesh,
        compiler_params=pltpu.CompilerParams(needs_layout_passes=False),
        out_type=jax.ShapeDtypeStruct((_NH, _NN), jnp.float32),
        scratch_types=[
            pltpu.VMEM((_ROWS * _NH,), jnp.float32),
            pltpu.VMEM((_CHUNK,), jnp.int32),
            pltpu.VMEM((_NH, _CHUNK), jnp.float32),
        ],
    )
    def body(table_hbm, idx_hbm, out_hbm, tab_v, idx_v, out_v):
        wid = lax.axis_index("s") * 2 + lax.axis_index("c")
        base = wid * _CHUNK
        pltpu.sync_copy(table_hbm, tab_v)
        pltpu.sync_copy(idx_hbm.at[pl.ds(base, _CHUNK)], idx_v)

        def step(t, carry):
            off = t * _L
            flat = idx_v[pl.ds(off, _L)] * _NH
            for h in range(_NH):
                out_v[h, pl.ds(off, _L)] = plsc.load_gather(tab_v, [flat + h])
            return carry

        lax.fori_loop(0, _CHUNK // _L, step, 0)
        pltpu.sync_copy(out_v, out_hbm.at[:, pl.ds(base, _CHUNK)])

    return body(table, idx_flat)


def kernel(table, rel_index):
    idx_flat = rel_index[:_N, :_N].reshape(-1)
    out = _sc_gather(table.reshape(-1), idx_flat)
    return out.reshape(1, _NH, _N, _N)


# speedup vs baseline: 4.4802x; 1.5841x over previous
import functools

import jax
import jax.numpy as jnp
from jax import lax
from jax.experimental import pallas as pl
from jax.experimental.pallas import tpu as pltpu
from jax.experimental.pallas import tpu_sc as plsc

_N = 256
_NH = 32
_NN = _N * _N
_ROWS = 961
_NW = 32
_CHUNK = _NN // _NW
_L = 16


def _sc_gather(table, idx_flat):
    mesh = plsc.VectorSubcoreMesh(core_axis_name="c", subcore_axis_name="s")

    @functools.partial(
        pl.kernel,
        mesh=mesh,
        compiler_params=pltpu.CompilerParams(needs_layout_passes=False),
        out_type=jax.ShapeDtypeStruct((_NH, _NN), jnp.float32),
        scratch_types=[
            pltpu.VMEM((_ROWS * _NH,), jnp.float32),
            pltpu.VMEM((_CHUNK,), jnp.int32),
            pltpu.VMEM((_NH, _CHUNK), jnp.float32),
        ],
    )
    def body(table_hbm, idx_hbm, out_hbm, tab_v, idx_v, out_v):
        wid = lax.axis_index("s") * 2 + lax.axis_index("c")
        base = wid * _CHUNK
        pltpu.sync_copy(table_hbm, tab_v)
        pltpu.sync_copy(idx_hbm.at[pl.ds(base, _CHUNK)], idx_v)

        def step(t, carry):
            off = t * _L
            ivec = idx_v[pl.ds(off, _L)]
            for h in range(_NH):
                out_v[h, pl.ds(off, _L)] = plsc.load_gather(
                    tab_v, [ivec + (h * _ROWS)]
                )
            return carry

        lax.fori_loop(0, _CHUNK // _L, step, 0)
        pltpu.sync_copy(out_v, out_hbm.at[:, pl.ds(base, _CHUNK)])

    return body(table, idx_flat)


def kernel(table, rel_index):
    idx_flat = rel_index[:_N, :_N].reshape(-1)
    out = _sc_gather(table.T.reshape(-1), idx_flat)
    return out.reshape(1, _NH, _N, _N)


# speedup vs baseline: 5.9687x; 1.3322x over previous
import functools

import jax
import jax.numpy as jnp
from jax import lax
from jax.experimental import pallas as pl
from jax.experimental.pallas import tpu as pltpu
from jax.experimental.pallas import tpu_sc as plsc

_N = 256
_NH = 32
_NN = _N * _N
_ROWS = 961
_NW = 32
_CHUNK = _NN // _NW
_L = 16


def _sc_gather(table, idx_flat):
    mesh = plsc.VectorSubcoreMesh(core_axis_name="c", subcore_axis_name="s")

    @functools.partial(
        pl.kernel,
        mesh=mesh,
        compiler_params=pltpu.CompilerParams(needs_layout_passes=False),
        out_type=jax.ShapeDtypeStruct((_NH, _NN), jnp.float32),
        scratch_types=[
            pltpu.VMEM((_ROWS * _NH,), jnp.float32),
            pltpu.VMEM((_CHUNK,), jnp.int32),
            pltpu.VMEM((_NH, _CHUNK), jnp.float32),
        ],
    )
    def body(table_hbm, idx_hbm, out_hbm, tab_v, idx_v, out_v):
        wid = lax.axis_index("s") * 2 + lax.axis_index("c")
        base = wid * _CHUNK
        pltpu.sync_copy(table_hbm, tab_v)
        pltpu.sync_copy(idx_hbm.at[pl.ds(base, _CHUNK)], idx_v)

        @plsc.parallel_loop(0, _CHUNK // _L, unroll=2)
        def step(t):
            off = t * _L
            ivec = idx_v[pl.ds(off, _L)]
            for h in range(_NH):
                out_v[h, pl.ds(off, _L)] = plsc.load_gather(
                    tab_v, [ivec + (h * _ROWS)]
                )
        pltpu.sync_copy(out_v, out_hbm.at[:, pl.ds(base, _CHUNK)])

    return body(table, idx_flat)


def kernel(table, rel_index):
    idx_flat = rel_index[:_N, :_N].reshape(-1)
    out = _sc_gather(table.T.reshape(-1), idx_flat)
    return out.reshape(1, _NH, _N, _N)
